# baseline (speedup 1.0000x reference)
import jax
import jax.numpy as jnp
from jax import lax
from jax.experimental import pallas as pl
from jax.experimental.pallas import tpu as pltpu, tpu_sc as plsc

NC, NS, L = 2, 16, 16
NW = NC * NS
B, G, D, N = 8, 8192, 64, 65536
SAMPLES_PER_W = (B * N) // NW
W_PER_BATCH = N // SAMPLES_PER_W
CHUNK = 256
NROW = CHUNK // 128
NCHUNKS = SAMPLES_PER_W // CHUNK
NVEC = CHUNK // L
OUTP = CHUNK + 1


def _body(pairs_hbm, grid_hbm, pos_hbm, out_hbm,
          grid_v, pos_v, idx_v, w_v, rows_v, out_t, sem_g, sem_o):
    wid = lax.axis_index("s") * NC + lax.axis_index("c")
    b = wid // W_PER_BATCH
    gbase = wid * SAMPLES_PER_W
    nbase = (wid % W_PER_BATCH) * SAMPLES_PER_W

    pltpu.sync_copy(grid_hbm.at[pl.ds(pl.multiple_of(b * G, G), G)], grid_v)
    pltpu.sync_copy(
        pos_hbm.at[pl.ds(pl.multiple_of(gbase, SAMPLES_PER_W), SAMPLES_PER_W)],
        pos_v)

    row_base = jnp.full((L,), b * G, jnp.int32)
    lane_iota = lax.iota(jnp.int32, L)

    def search_chunk(c, buf):
        @plsc.parallel_loop(0, NVEC)
        def _(i):
            pos = pos_v[pl.ds(c * CHUNK + i * L, L)]
            idx = (pos * 0.0).astype(jnp.int32)
            bit = G // 2
            while bit > 0:
                j = idx + bit
                g = plsc.load_gather(grid_v, [j])
                idx = jnp.where(g <= pos, j, idx)
                bit //= 2
            idx = jnp.minimum(idx, G - 2)
            gl = plsc.load_gather(grid_v, [idx])
            gr = plsc.load_gather(grid_v, [idx + 1])
            wr = jnp.clip((pos - gl) / jnp.maximum(gr - gl, 1e-8), 0.0, 1.0)
            r = i // (128 // L)
            col = (i % (128 // L)) * L
            idx_v[buf, r, pl.ds(col, L)] = row_base + idx
            w_v[pl.ds(buf * CHUNK + i * L, L)] = wr

    def gather_descs(buf):
        return [
            pltpu.make_async_copy(
                pairs_hbm.at[idx_v.at[buf, r]],
                rows_v.at[buf, pl.ds(r * 128, 128)], sem_g)
            for r in range(NROW)
        ]

    def lerp_chunk(buf):
        @plsc.parallel_loop(0, CHUNK)
        def _(n):
            wrv = plsc.load_gather(
                w_v, [jnp.full((L,), buf * CHUNK + n, jnp.int32)])
            wlv = 1.0 - wrv
            nz = n * 0
            for dc in range(D // L):
                fl = rows_v[buf, n, pl.ds(dc * L, L)]
                fr = rows_v[buf, n, pl.ds(D + dc * L, L)]
                val = wlv * fl + wrv * fr
                rows16 = lane_iota + (nz + dc * L)
                plsc.store_scatter(
                    out_t, [rows16, jnp.full((L,), n, jnp.int32)], val)

    def out_desc(c):
        off = pl.multiple_of(nbase + c * CHUNK, CHUNK)
        return pltpu.make_async_copy(
            out_t.at[:, pl.ds(0, CHUNK)],
            out_hbm.at[pl.ds(pl.multiple_of(b * D, D), D), pl.ds(off, CHUNK)],
            sem_o)

    search_chunk(0, 0)
    for cp in gather_descs(0):
        cp.start()

    def chunk_body(c, _):
        buf = lax.rem(c, 2)
        nbuf = lax.rem(c + 1, 2)

        @pl.when(c < NCHUNKS - 1)
        def _():
            search_chunk(c + 1, nbuf)
            for cp in gather_descs(nbuf):
                cp.start()

        for cp in gather_descs(buf):
            cp.wait()
        @pl.when(c > 0)
        def _():
            out_desc(c - 1).wait()
        lerp_chunk(buf)
        out_desc(c).start()
        return 0

    lax.fori_loop(0, NCHUNKS, chunk_body, 0)
    out_desc(NCHUNKS - 1).wait()


@jax.jit
def kernel(field, grid_points, sample_positions):
    shifted = jnp.concatenate([field[:, 1:, :], field[:, -1:, :]], axis=1)
    pairs = jnp.concatenate([field, shifted], axis=2).reshape(B * G, 2 * D)
    grid_flat = grid_points.reshape(B * G)
    pos_flat = sample_positions.reshape(B * N)

    mesh = plsc.VectorSubcoreMesh(
        core_axis_name="c", subcore_axis_name="s",
        num_cores=NC, num_subcores=NS)
    out_t = pl.kernel(
        _body,
        out_type=jax.ShapeDtypeStruct((B * D, N), jnp.float32),
        mesh=mesh,
        scratch_types=[
            pltpu.VMEM((G,), jnp.float32),
            pltpu.VMEM((SAMPLES_PER_W,), jnp.float32),
            pltpu.VMEM((2, NROW, 128), jnp.int32),
            pltpu.VMEM((2 * CHUNK,), jnp.float32),
            pltpu.VMEM((2, CHUNK, 2 * D), jnp.float32),
            pltpu.VMEM((D, OUTP), jnp.float32),
            pltpu.SemaphoreType.DMA,
            pltpu.SemaphoreType.DMA,
        ],
        compiler_params=pltpu.CompilerParams(needs_layout_passes=False),
    )(pairs, grid_flat, pos_flat)
    return out_t.reshape(B, D, N).transpose(0, 2, 1)


# speedup vs baseline: 1.0005x; 1.0005x over previous
import jax
import jax.numpy as jnp
from jax import lax
from jax.experimental import pallas as pl
from jax.experimental.pallas import tpu as pltpu, tpu_sc as plsc

NC, NS, L = 2, 16, 16
NW = NC * NS
B, G, D, N = 8, 8192, 64, 65536
SAMPLES_PER_W = (B * N) // NW
W_PER_BATCH = N // SAMPLES_PER_W
CHUNK = 256
NROW = CHUNK // 128
NCHUNKS = SAMPLES_PER_W // CHUNK
NVEC = CHUNK // L
OUTP = CHUNK + 8


def _body(pairs_hbm, grid_hbm, pos_hbm, out_hbm,
          grid_v, pos_v, idx_v, w_v, rows_v, out_t, sem_g, sem_o):
    wid = lax.axis_index("s") * NC + lax.axis_index("c")
    b = wid // W_PER_BATCH
    gbase = wid * SAMPLES_PER_W
    nbase = (wid % W_PER_BATCH) * SAMPLES_PER_W

    pltpu.sync_copy(grid_hbm.at[pl.ds(pl.multiple_of(b * G, G), G)], grid_v)
    pltpu.sync_copy(
        pos_hbm.at[pl.ds(pl.multiple_of(gbase, SAMPLES_PER_W), SAMPLES_PER_W)],
        pos_v)

    row_base = jnp.full((L,), b * G, jnp.int32)
    lane_iota = lax.iota(jnp.int32, L)

    def search_chunk(c, buf):
        @plsc.parallel_loop(0, NVEC)
        def _(i):
            pos = pos_v[pl.ds(c * CHUNK + i * L, L)]
            idx = (pos * 0.0).astype(jnp.int32)
            bit = G // 2
            while bit > 0:
                j = idx + bit
                g = plsc.load_gather(grid_v, [j])
                idx = jnp.where(g <= pos, j, idx)
                bit //= 2
            idx = jnp.minimum(idx, G - 2)
            gl = plsc.load_gather(grid_v, [idx])
            gr = plsc.load_gather(grid_v, [idx + 1])
            wr = jnp.clip((pos - gl) / jnp.maximum(gr - gl, 1e-8), 0.0, 1.0)
            r = i // (128 // L)
            col = (i % (128 // L)) * L
            idx_v[buf, r, pl.ds(col, L)] = row_base + idx
            w_v[pl.ds(buf * CHUNK + i * L, L)] = wr

    def gather_descs(buf):
        return [
            pltpu.make_async_copy(
                pairs_hbm.at[idx_v.at[buf, r]],
                rows_v.at[buf, pl.ds(r * 128, 128)], sem_g)
            for r in range(NROW)
        ]

    def lerp_chunk(buf):
        @plsc.parallel_loop(0, CHUNK)
        def _(n):
            wrv = plsc.load_gather(
                w_v, [jnp.full((L,), buf * CHUNK + n, jnp.int32)])
            wlv = 1.0 - wrv
            nz = n * 0
            for dc in range(D // L):
                fl = rows_v[buf, n, pl.ds(dc * L, L)]
                fr = rows_v[buf, n, pl.ds(D + dc * L, L)]
                val = wlv * fl + wrv * fr
                rows16 = lane_iota + (nz + dc * L)
                plsc.store_scatter(
                    out_t, [rows16, jnp.full((L,), n, jnp.int32)], val)

    def out_desc(c):
        off = pl.multiple_of(nbase + c * CHUNK, CHUNK)
        return pltpu.make_async_copy(
            out_t.at[:, pl.ds(0, CHUNK)],
            out_hbm.at[pl.ds(pl.multiple_of(b * D, D), D), pl.ds(off, CHUNK)],
            sem_o)

    search_chunk(0, 0)
    for cp in gather_descs(0):
        cp.start()

    def chunk_body(c, _):
        buf = lax.rem(c, 2)
        nbuf = lax.rem(c + 1, 2)

        @pl.when(c < NCHUNKS - 1)
        def _():
            search_chunk(c + 1, nbuf)
            for cp in gather_descs(nbuf):
                cp.start()

        for cp in gather_descs(buf):
            cp.wait()
        @pl.when(c > 0)
        def _():
            out_desc(c - 1).wait()
        lerp_chunk(buf)
        out_desc(c).start()
        return 0

    lax.fori_loop(0, NCHUNKS, chunk_body, 0)
    out_desc(NCHUNKS - 1).wait()


@jax.jit
def kernel(field, grid_points, sample_positions):
    shifted = jnp.concatenate([field[:, 1:, :], field[:, -1:, :]], axis=1)
    pairs = jnp.concatenate([field, shifted], axis=2).reshape(B * G, 2 * D)
    grid_flat = grid_points.reshape(B * G)
    pos_flat = sample_positions.reshape(B * N)

    mesh = plsc.VectorSubcoreMesh(
        core_axis_name="c", subcore_axis_name="s",
        num_cores=NC, num_subcores=NS)
    out_t = pl.kernel(
        _body,
        out_type=jax.ShapeDtypeStruct((B * D, N), jnp.float32),
        mesh=mesh,
        scratch_types=[
            pltpu.VMEM((G,), jnp.float32),
            pltpu.VMEM((SAMPLES_PER_W,), jnp.float32),
            pltpu.VMEM((2, NROW, 128), jnp.int32),
            pltpu.VMEM((2 * CHUNK,), jnp.float32),
            pltpu.VMEM((2, CHUNK, 2 * D), jnp.float32),
            pltpu.VMEM((D, OUTP), jnp.float32),
            pltpu.SemaphoreType.DMA,
            pltpu.SemaphoreType.DMA,
        ],
        compiler_params=pltpu.CompilerParams(needs_layout_passes=False),
    )(pairs, grid_flat, pos_flat)
    return out_t.reshape(B, D, N).transpose(0, 2, 1)


# speedup vs baseline: 1.6868x; 1.6859x over previous
import jax
import jax.numpy as jnp
from jax import lax
from jax.experimental import pallas as pl
from jax.experimental.pallas import tpu as pltpu, tpu_sc as plsc

NC, NS, L = 2, 16, 16
NW = NC * NS
B, G, D, N = 8, 8192, 64, 65536
SAMPLES_PER_W = (B * N) // NW
W_PER_BATCH = N // SAMPLES_PER_W
CHUNK = 256
NROW = CHUNK // 128
NCHUNKS = SAMPLES_PER_W // CHUNK
NVEC = CHUNK // L


def _body(pairs_hbm, grid_hbm, pos_hbm, out_hbm,
          grid_v, pos_v, idx_v, w_v, rows_v, out_t, sem_g, sem_o):
    wid = lax.axis_index("s") * NC + lax.axis_index("c")
    b = wid // W_PER_BATCH
    gbase = wid * SAMPLES_PER_W

    pltpu.sync_copy(grid_hbm.at[pl.ds(pl.multiple_of(b * G, G), G)], grid_v)
    pltpu.sync_copy(
        pos_hbm.at[pl.ds(pl.multiple_of(gbase, SAMPLES_PER_W), SAMPLES_PER_W)],
        pos_v)

    row_base = jnp.full((L,), b * G, jnp.int32)

    def search_chunk(c, buf):
        @plsc.parallel_loop(0, NVEC)
        def _(i):
            pos = pos_v[pl.ds(c * CHUNK + i * L, L)]
            idx = (pos * 0.0).astype(jnp.int32)
            bit = G // 2
            while bit > 0:
                j = idx + bit
                g = plsc.load_gather(grid_v, [j])
                idx = jnp.where(g <= pos, j, idx)
                bit //= 2
            idx = jnp.minimum(idx, G - 2)
            gl = plsc.load_gather(grid_v, [idx])
            gr = plsc.load_gather(grid_v, [idx + 1])
            wr = jnp.clip((pos - gl) / jnp.maximum(gr - gl, 1e-8), 0.0, 1.0)
            r = i // (128 // L)
            col = (i % (128 // L)) * L
            idx_v[buf, r, pl.ds(col, L)] = row_base + idx
            w_v[pl.ds(buf * CHUNK + i * L, L)] = wr

    def gather_descs(buf):
        return [
            pltpu.make_async_copy(
                pairs_hbm.at[idx_v.at[buf, r]],
                rows_v.at[buf, pl.ds(r * 128, 128)], sem_g)
            for r in range(NROW)
        ]

    def lerp_chunk(buf):
        @plsc.parallel_loop(0, CHUNK)
        def _(n):
            wrv = plsc.load_gather(
                w_v, [jnp.full((L,), buf * CHUNK + n, jnp.int32)])
            wlv = 1.0 - wrv
            for dc in range(D // L):
                fl = rows_v[buf, n, pl.ds(dc * L, L)]
                fr = rows_v[buf, n, pl.ds(D + dc * L, L)]
                out_t[n, pl.ds(dc * L, L)] = wlv * fl + wrv * fr

    def out_desc(c):
        off = pl.multiple_of(gbase + c * CHUNK, CHUNK)
        return pltpu.make_async_copy(
            out_t, out_hbm.at[pl.ds(off, CHUNK)], sem_o)

    search_chunk(0, 0)
    for cp in gather_descs(0):
        cp.start()

    def chunk_body(c, _):
        buf = lax.rem(c, 2)
        nbuf = lax.rem(c + 1, 2)

        @pl.when(c < NCHUNKS - 1)
        def _():
            search_chunk(c + 1, nbuf)
            for cp in gather_descs(nbuf):
                cp.start()

        for cp in gather_descs(buf):
            cp.wait()
        @pl.when(c > 0)
        def _():
            out_desc(c - 1).wait()
        lerp_chunk(buf)
        out_desc(c).start()
        return 0

    lax.fori_loop(0, NCHUNKS, chunk_body, 0)
    out_desc(NCHUNKS - 1).wait()


@jax.jit
def kernel(field, grid_points, sample_positions):
    shifted = jnp.concatenate([field[:, 1:, :], field[:, -1:, :]], axis=1)
    pairs = jnp.concatenate([field, shifted], axis=2).reshape(B * G, 2 * D)
    grid_flat = grid_points.reshape(B * G)
    pos_flat = sample_positions.reshape(B * N)

    mesh = plsc.VectorSubcoreMesh(
        core_axis_name="c", subcore_axis_name="s",
        num_cores=NC, num_subcores=NS)
    out_t = pl.kernel(
        _body,
        out_type=jax.ShapeDtypeStruct((B * N, D), jnp.float32),
        mesh=mesh,
        scratch_types=[
            pltpu.VMEM((G,), jnp.float32),
            pltpu.VMEM((SAMPLES_PER_W,), jnp.float32),
            pltpu.VMEM((2, NROW, 128), jnp.int32),
            pltpu.VMEM((2 * CHUNK,), jnp.float32),
            pltpu.VMEM((2, CHUNK, 2 * D), jnp.float32),
            pltpu.VMEM((CHUNK, D), jnp.float32),
            pltpu.SemaphoreType.DMA,
            pltpu.SemaphoreType.DMA,
        ],
        compiler_params=pltpu.CompilerParams(needs_layout_passes=False),
    )(pairs, grid_flat, pos_flat)
    return out_t.reshape(B, N, D)
